# baseline (speedup 1.0000x reference)
import functools

import jax
import jax.numpy as jnp
from jax import lax
from jax.experimental import pallas as pl
from jax.experimental.pallas import tpu as pltpu

_HEAD_DIM = 128
_HALF = _HEAD_DIM // 2
_ROPE_THETA = 500000.0
_SEQ = 8192
_BLOCK = 512
_NBLK = _SEQ // _BLOCK
_CHUNK = 8


def _compute_block(p0f):
    shape = (1, _CHUNK, _HEAD_DIM)
    col = lax.broadcasted_iota(jnp.int32, shape, 2)
    j = jnp.bitwise_and(col, _HALF - 1).astype(jnp.float32)
    inv_freq = jnp.exp(j * (-2.0 / _HEAD_DIM * jnp.log(_ROPE_THETA)))
    r = lax.broadcasted_iota(jnp.int32, shape, 1).astype(jnp.float32)
    ang = (p0f + r) * inv_freq
    c0 = jnp.cos(ang)
    s0 = jnp.sin(ang)
    rc = jnp.cos(jnp.float32(_CHUNK) * inv_freq)
    rs = jnp.sin(jnp.float32(_CHUNK) * inv_freq)
    chunks = [(c0, s0, 0)]
    stride = _CHUNK
    while stride < _BLOCK:
        new = [(c * rc - s * rs, s * rc + c * rs, off + stride)
               for (c, s, off) in chunks]
        chunks = chunks + new
        stride *= 2
        if stride < _BLOCK:
            rc, rs = rc * rc - rs * rs, 2.0 * rc * rs
    return chunks


def _rope_kernel(cos_hbm, sin_hbm, cvm, svm, sems):
    for b in range(_NBLK):
        base = b * _BLOCK
        for c, s, off in _compute_block(jnp.float32(base)):
            cvm[:, pl.ds(base + off, _CHUNK), :] = c
            svm[:, pl.ds(base + off, _CHUNK), :] = s
        sl = pl.ds(base, _BLOCK)
        pltpu.make_async_copy(
            cvm.at[:, sl, :], cos_hbm.at[:, sl, :], sems.at[2 * b]).start()
        pltpu.make_async_copy(
            svm.at[:, sl, :], sin_hbm.at[:, sl, :], sems.at[2 * b + 1]).start()
    for b in range(_NBLK):
        sl = pl.ds(b * _BLOCK, _BLOCK)
        pltpu.make_async_copy(
            cvm.at[:, sl, :], cos_hbm.at[:, sl, :], sems.at[2 * b]).wait()
        pltpu.make_async_copy(
            svm.at[:, sl, :], sin_hbm.at[:, sl, :], sems.at[2 * b + 1]).wait()


@functools.partial(jax.jit, static_argnums=())
def kernel(x, W):
    del x, W
    out_shape = jax.ShapeDtypeStruct((1, _SEQ, _HEAD_DIM), jnp.float32)
    anyspec = pl.BlockSpec(memory_space=pl.ANY)
    cos, sin = pl.pallas_call(
        _rope_kernel,
        out_specs=(anyspec, anyspec),
        out_shape=(out_shape, out_shape),
        scratch_shapes=[
            pltpu.VMEM((1, _SEQ, _HEAD_DIM), jnp.float32),
            pltpu.VMEM((1, _SEQ, _HEAD_DIM), jnp.float32),
            pltpu.SemaphoreType.DMA((2 * _NBLK,)),
        ],
    )()
    return cos, sin


# speedup vs baseline: 1.0297x; 1.0297x over previous
import functools

import jax
import jax.numpy as jnp
from jax import lax
from jax.experimental import pallas as pl
from jax.experimental.pallas import tpu as pltpu

_HEAD_DIM = 128
_HALF = _HEAD_DIM // 2
_ROPE_THETA = 500000.0
_SEQ = 8192
_BLOCKS = (256, 256, 512, 1024, 1024, 1024, 1024, 1024, 1024, 1024)
_CHUNK = 8


def _compute_block(p0f, block):
    shape = (1, _CHUNK, _HEAD_DIM)
    col = lax.broadcasted_iota(jnp.int32, shape, 2)
    j = jnp.bitwise_and(col, _HALF - 1).astype(jnp.float32)
    inv_freq = jnp.exp(j * (-2.0 / _HEAD_DIM * jnp.log(_ROPE_THETA)))
    r = lax.broadcasted_iota(jnp.int32, shape, 1).astype(jnp.float32)
    ang = (p0f + r) * inv_freq
    c0 = jnp.cos(ang)
    s0 = jnp.sin(ang)
    rc = jnp.cos(jnp.float32(_CHUNK) * inv_freq)
    rs = jnp.sin(jnp.float32(_CHUNK) * inv_freq)
    chunks = [(c0, s0, 0)]
    stride = _CHUNK
    while stride < block:
        new = [(c * rc - s * rs, s * rc + c * rs, off + stride)
               for (c, s, off) in chunks]
        chunks = chunks + new
        stride *= 2
        if stride < block:
            rc, rs = rc * rc - rs * rs, 2.0 * rc * rs
    return chunks


def _rope_kernel(cos_hbm, sin_hbm, cvm, svm, sems):
    base = 0
    for b, block in enumerate(_BLOCKS):
        for c, s, off in _compute_block(jnp.float32(base), block):
            cvm[:, pl.ds(base + off, _CHUNK), :] = c
            svm[:, pl.ds(base + off, _CHUNK), :] = s
        sl = pl.ds(base, block)
        pltpu.make_async_copy(
            cvm.at[:, sl, :], cos_hbm.at[:, sl, :], sems.at[2 * b]).start()
        pltpu.make_async_copy(
            svm.at[:, sl, :], sin_hbm.at[:, sl, :], sems.at[2 * b + 1]).start()
        base += block
    base = 0
    for b, block in enumerate(_BLOCKS):
        sl = pl.ds(base, block)
        pltpu.make_async_copy(
            cvm.at[:, sl, :], cos_hbm.at[:, sl, :], sems.at[2 * b]).wait()
        pltpu.make_async_copy(
            svm.at[:, sl, :], sin_hbm.at[:, sl, :], sems.at[2 * b + 1]).wait()
        base += block


@functools.partial(jax.jit, static_argnums=())
def kernel(x, W):
    del x, W
    out_shape = jax.ShapeDtypeStruct((1, _SEQ, _HEAD_DIM), jnp.float32)
    anyspec = pl.BlockSpec(memory_space=pl.ANY)
    cos, sin = pl.pallas_call(
        _rope_kernel,
        out_specs=(anyspec, anyspec),
        out_shape=(out_shape, out_shape),
        scratch_shapes=[
            pltpu.VMEM((1, _SEQ, _HEAD_DIM), jnp.float32),
            pltpu.VMEM((1, _SEQ, _HEAD_DIM), jnp.float32),
            pltpu.SemaphoreType.DMA((2 * len(_BLOCKS),)),
        ],
    )()
    return cos, sin
